# baseline (speedup 1.0000x reference)
import jax
import jax.numpy as jnp
from jax import lax
from jax.experimental import pallas as pl
from jax.experimental.pallas import tpu as pltpu
from jax.experimental.pallas import tpu_sc as plsc

B = 1024
SEQ = 200
EMB = 64
NC = 2
NS = 16
NW = NC * NS
BT = B // 128
NU = SEQ * BT
UPW = NU // NW
NBUF = 4
PRE = 2
LANES = 16
ET = EMB // 8


def _unit(u0, cur, row0):
    u = u0 + cur
    s = u // BT
    bt = lax.rem(u, BT)
    return s, bt, s - row0


def _gather_refs(qtab, ctab, qi, ci, q_buf, c_buf, lr, bt, slot):
    start = pl.multiple_of(128 * bt, 128)
    return (
        (qtab.at[qi.at[lr, pl.ds(start, 128)]], q_buf.at[slot]),
        (ctab.at[ci.at[lr, pl.ds(start, 128)]], c_buf.at[slot]),
    )


def _body(qT_hbm, cT_hbm, qtab_hbm, ctab_hbm, pos_hbm, out_hbm,
          qi, ci, pos_v, q_buf, c_buf, o_buf, gsem, osem):
    wid = lax.axis_index("s") * NC + lax.axis_index("c")
    u0 = wid * UPW
    row0 = jnp.minimum(u0 // BT, SEQ - 8)
    pltpu.sync_copy(qT_hbm.at[pl.ds(row0, 8)], qi)
    pltpu.sync_copy(cT_hbm.at[pl.ds(row0, 8)], ci)
    pltpu.sync_copy(pos_hbm.at[pl.ds(row0, 8)], pos_v)

    def fire(cur, slot):
        _, bt, lr = _unit(u0, cur, row0)
        for src, dst in _gather_refs(qtab_hbm, ctab_hbm, qi, ci,
                                     q_buf, c_buf, lr, bt, slot):
            pltpu.async_copy(src, dst, gsem[slot])

    def wait_gathers(cur, slot):
        _, bt, lr = _unit(u0, cur, row0)
        for src, dst in _gather_refs(qtab_hbm, ctab_hbm, qi, ci,
                                     q_buf, c_buf, lr, bt, slot):
            pltpu.make_async_copy(src, dst, gsem[slot]).wait()

    def compute(cur, slot):
        _, _, lr = _unit(u0, cur, row0)
        iota = lax.iota(jnp.int32, LANES)
        psg = [pos_v[lr, pl.ds(g * LANES, LANES)] for g in range(EMB // LANES)]

        @pl.loop(0, 8)
        def _bg(bg):
            rows = bg * LANES + iota
            for E in range(ET):
                for e in range(8):
                    col = 8 * E + e
                    cols = jnp.full((LANES,), col, jnp.int32)
                    qv = plsc.load_gather(q_buf.at[slot], [rows, cols])
                    cv = plsc.load_gather(c_buf.at[slot], [rows, cols])
                    ps = psg[col // LANES][col % LANES]
                    o_buf[slot, E, e, pl.ds(bg * LANES, LANES)] = qv + cv + ps

    def fire_write(cur, slot):
        s, bt, _ = _unit(u0, cur, row0)
        pltpu.async_copy(o_buf.at[slot], out_hbm.at[s, :, bt], osem[slot])

    def wait_write(cur, slot):
        s, bt, _ = _unit(u0, cur, row0)
        pltpu.make_async_copy(o_buf.at[slot], out_hbm.at[s, :, bt],
                              osem[slot]).wait()

    for j in range(PRE):
        fire(j, j % NBUF)

    LOOPED = (UPW // NBUF) * NBUF

    @pl.loop(0, LOOPED, step=NBUF)
    def _turns(k):
        for b in range(NBUF):
            cur = k + b
            wait_gathers(cur, b)
            compute(cur, b)
            fire_write(cur, b)
            nxt = cur + PRE
            bn = (b + PRE) % NBUF

            @pl.when(nxt < UPW)
            def _():
                @pl.when(cur >= NBUF - PRE)
                def _():
                    wait_write(nxt - NBUF, bn)
                fire(nxt, bn)

    for cur in range(LOOPED, UPW):
        b = cur % NBUF
        wait_gathers(cur, b)
        compute(cur, b)
        fire_write(cur, b)

    for cur in range(UPW - NBUF, UPW):
        wait_write(cur, cur % NBUF)


def kernel(questions, category, question_table, category_table, pos_table):
    out5d = pl.kernel(
        _body,
        out_type=jax.ShapeDtypeStruct((SEQ, ET, BT, 8, 128), jnp.float32),
        mesh=plsc.VectorSubcoreMesh(core_axis_name="c", subcore_axis_name="s"),
        compiler_params=pltpu.CompilerParams(use_tc_tiling_on_sc=False,
                                             needs_layout_passes=False),
        scratch_types=[
            pltpu.VMEM((8, B), jnp.int32),
            pltpu.VMEM((8, B), jnp.int32),
            pltpu.VMEM((8, EMB), jnp.float32),
            pltpu.VMEM((NBUF, 128, EMB), jnp.float32),
            pltpu.VMEM((NBUF, 128, EMB), jnp.float32),
            pltpu.VMEM((NBUF, ET, 8, 128), jnp.float32),
            [pltpu.SemaphoreType.DMA] * NBUF,
            [pltpu.SemaphoreType.DMA] * NBUF,
        ],
    )(questions.astype(jnp.int32).T, category.astype(jnp.int32).T,
      question_table, category_table, pos_table)
    return out5d.transpose(2, 4, 0, 1, 3).reshape(B, SEQ, EMB)


# speedup vs baseline: 2.7134x; 2.7134x over previous
import jax
import jax.numpy as jnp
from jax import lax
from jax.experimental import pallas as pl
from jax.experimental.pallas import tpu as pltpu
from jax.experimental.pallas import tpu_sc as plsc

B = 1024
SEQ = 200
EMB = 64
NC = 2
NS = 16
NW = NC * NS
BT = B // 128
NU = SEQ * BT
UPW = NU // NW
NBUF = 4
PRE = 2
LANES = 16
ET = EMB // 8


def _unit(u0, cur, row0):
    u = u0 + cur
    s = u // BT
    bt = lax.rem(u, BT)
    return s, bt, s - row0


def _gather_refs(qtab, ctab, qi, ci, q_buf, c_buf, lr, bt, slot):
    start = pl.multiple_of(128 * bt, 128)
    return (
        (qtab.at[qi.at[lr, pl.ds(start, 128)]], q_buf.at[slot]),
        (ctab.at[ci.at[lr, pl.ds(start, 128)]], c_buf.at[slot]),
    )


def _body(qT_hbm, cT_hbm, qtab_hbm, ctab_hbm, pos_hbm, out_hbm,
          qi, ci, pos_v, q_buf, c_buf, o_buf, gsem, osem):
    wid = lax.axis_index("s") * NC + lax.axis_index("c")
    u0 = wid * UPW
    row0 = jnp.minimum(u0 // BT, SEQ - 8)
    pltpu.sync_copy(qT_hbm.at[pl.ds(row0, 8)], qi)
    pltpu.sync_copy(cT_hbm.at[pl.ds(row0, 8)], ci)
    pltpu.sync_copy(pos_hbm.at[pl.ds(row0, 8)], pos_v)

    def fire(cur, slot):
        _, bt, lr = _unit(u0, cur, row0)
        for src, dst in _gather_refs(qtab_hbm, ctab_hbm, qi, ci,
                                     q_buf, c_buf, lr, bt, slot):
            pltpu.async_copy(src, dst, gsem[slot])

    def wait_gathers(cur, slot):
        _, bt, lr = _unit(u0, cur, row0)
        for src, dst in _gather_refs(qtab_hbm, ctab_hbm, qi, ci,
                                     q_buf, c_buf, lr, bt, slot):
            pltpu.make_async_copy(src, dst, gsem[slot]).wait()

    def compute(cur, slot):
        _, _, lr = _unit(u0, cur, row0)
        iota = lax.iota(jnp.int32, LANES)
        psg = [pos_v[lr, pl.ds(g * LANES, LANES)] for g in range(EMB // LANES)]
        eidx = [((g * LANES + iota) // 8, (g * LANES + iota) % 8)
                for g in range(EMB // LANES)]

        @pl.loop(0, 128, unroll=2)
        def _item(i):
            bb = jnp.full((LANES,), i, jnp.int32)
            for g in range(EMB // LANES):
                qv = q_buf[slot, i, pl.ds(g * LANES, LANES)]
                cv = c_buf[slot, i, pl.ds(g * LANES, LANES)]
                plsc.store_scatter(o_buf.at[slot], [eidx[g][0], eidx[g][1], bb],
                                   qv + cv + psg[g])

    def fire_write(cur, slot):
        s, bt, _ = _unit(u0, cur, row0)
        pltpu.async_copy(o_buf.at[slot, :, :, pl.ds(0, 128)],
                         out_hbm.at[s, :, bt], osem[slot])

    def wait_write(cur, slot):
        s, bt, _ = _unit(u0, cur, row0)
        pltpu.make_async_copy(o_buf.at[slot, :, :, pl.ds(0, 128)],
                              out_hbm.at[s, :, bt], osem[slot]).wait()

    for j in range(PRE):
        fire(j, j % NBUF)

    LOOPED = (UPW // NBUF) * NBUF

    @pl.loop(0, LOOPED, step=NBUF)
    def _turns(k):
        for b in range(NBUF):
            cur = k + b
            wait_gathers(cur, b)
            compute(cur, b)
            fire_write(cur, b)
            nxt = cur + PRE
            bn = (b + PRE) % NBUF

            @pl.when(nxt < UPW)
            def _():
                @pl.when(cur >= NBUF - PRE)
                def _():
                    wait_write(nxt - NBUF, bn)
                fire(nxt, bn)

    for cur in range(LOOPED, UPW):
        b = cur % NBUF
        wait_gathers(cur, b)
        compute(cur, b)
        fire_write(cur, b)

    for cur in range(UPW - NBUF, UPW):
        wait_write(cur, cur % NBUF)


def kernel(questions, category, question_table, category_table, pos_table):
    out5d = pl.kernel(
        _body,
        out_type=jax.ShapeDtypeStruct((SEQ, ET, BT, 8, 128), jnp.float32),
        mesh=plsc.VectorSubcoreMesh(core_axis_name="c", subcore_axis_name="s"),
        compiler_params=pltpu.CompilerParams(use_tc_tiling_on_sc=False,
                                             needs_layout_passes=False),
        scratch_types=[
            pltpu.VMEM((8, B), jnp.int32),
            pltpu.VMEM((8, B), jnp.int32),
            pltpu.VMEM((8, EMB), jnp.float32),
            pltpu.VMEM((NBUF, 128, EMB), jnp.float32),
            pltpu.VMEM((NBUF, 128, EMB), jnp.float32),
            pltpu.VMEM((NBUF, ET, 8, 129), jnp.float32),
            [pltpu.SemaphoreType.DMA] * NBUF,
            [pltpu.SemaphoreType.DMA] * NBUF,
        ],
    )(questions.astype(jnp.int32).T, category.astype(jnp.int32).T,
      question_table, category_table, pos_table)
    return out5d.transpose(2, 4, 0, 1, 3).reshape(B, SEQ, EMB)
